# baseline (speedup 1.0000x reference)
import jax
import jax.numpy as jnp
import numpy as np
from jax import lax
from jax.experimental import pallas as pl
from jax.experimental.pallas import tpu as pltpu
from jax.experimental.pallas import tpu_sc as plsc

_PSI = 1.0
_MULT = 20.0
_EPS = 1e-5

_B = 16384
_D = 128
_IMG = 512
_IMGP = _IMG // 2
_NW = 32
_BPW = _B // _NW
_C = 32
_NCH = _BPW // _C
_NBUF = 2

_BT = 512
_NT = _B // _BT

_SET_KEYS = ("ets", "eto", "ebs", "ebo", "rall", "imgfs", "imgfo",
             "si", "so")

def _sc_body(s_h, r_h, o_h, et_h, rall_h, eb_h, img_h,
             gs_h, go_h, ps_h, po_h, b3_h,
             sidx, ridx, oidx, *bufs):
    wid = lax.axis_index("sub") * 2 + lax.axis_index("core")
    base = wid * _BPW
    pltpu.sync_copy(s_h.at[pl.ds(base, _BPW)], sidx)
    pltpu.sync_copy(r_h.at[pl.ds(base, _BPW)], ridx)
    pltpu.sync_copy(o_h.at[pl.ds(base, _BPW)], oidx)

    nk = len(_SET_KEYS)
    sets = [dict(zip(_SET_KEYS, bufs[i * nk:(i + 1) * nk]))
            for i in range(_NBUF)]

    def fire_in(c, st):
        sl = pl.ds(c * _C, _C)
        cs, cr, co = sidx.at[sl], ridx.at[sl], oidx.at[sl]
        return [
            pltpu.async_copy(et_h.at[cs], st["ets"], st["si"]),
            pltpu.async_copy(et_h.at[co], st["eto"], st["si"]),
            pltpu.async_copy(eb_h.at[cs], st["ebs"], st["si"]),
            pltpu.async_copy(eb_h.at[co], st["ebo"], st["si"]),
            pltpu.async_copy(rall_h.at[cr], st["rall"], st["si"]),
            pltpu.async_copy(img_h.at[cs], st["imgfs"], st["si"]),
            pltpu.async_copy(img_h.at[co], st["imgfo"], st["si"]),
        ]

    def compute(st):
        ets, eto, ebs, ebo, rall = (st["ets"], st["eto"], st["ebs"],
                                    st["ebo"], st["rall"])
        def prow(i, cc):
            for j in range(_D // 16):
                ix = (i, pl.ds(j * 16, 16))
                ets[ix] = ets[ix] * rall[i, pl.ds(j * 16, 16)]
                eto[ix] = eto[ix] * rall[i, pl.ds(_D + j * 16, 16)]
                ebs[ix] = (ebs[ix] * rall[i, pl.ds(2 * _D + j * 16, 16)]
                           * ebo[ix])
            return cc

        lax.fori_loop(0, _C, prow, 0)

    def fire_out(c, st):
        sl = pl.ds(base + c * _C, _C)
        return [
            pltpu.async_copy(st["ets"], ps_h.at[sl], st["so"]),
            pltpu.async_copy(st["eto"], po_h.at[sl], st["so"]),
            pltpu.async_copy(st["ebs"], b3_h.at[sl], st["so"]),
            pltpu.async_copy(st["imgfs"], gs_h.at[sl], st["so"]),
            pltpu.async_copy(st["imgfo"], go_h.at[sl], st["so"]),
        ]

    in_fl = {0: fire_in(0, sets[0])}
    out_fl = [[], []]
    for c in range(_NCH):
        b = c % _NBUF
        if c + 1 < _NCH:
            nb = (c + 1) % _NBUF
            for cp in out_fl[nb]:
                cp.wait()
            out_fl[nb] = []
            in_fl[c + 1] = fire_in(c + 1, sets[nb])
        for cp in in_fl.pop(c):
            cp.wait()
        compute(sets[b])
        out_fl[b] = fire_out(c, sets[b])
    for lst in out_fl:
        for cp in lst:
            cp.wait()


def _sc_gather(*args):
    set_types = [
        pltpu.VMEM((_C, _D), jnp.float32),
        pltpu.VMEM((_C, _D), jnp.float32),
        pltpu.VMEM((_C, _D), jnp.float32),
        pltpu.VMEM((_C, _D), jnp.float32),
        pltpu.VMEM((_C, 3 * _D), jnp.float32),
        pltpu.VMEM((_C, _IMG), jnp.float32),
        pltpu.VMEM((_C, _IMG), jnp.float32),
        pltpu.SemaphoreType.DMA,
        pltpu.SemaphoreType.DMA,
    ]
    fn = pl.kernel(
        _sc_body,
        mesh=plsc.VectorSubcoreMesh(core_axis_name="core", subcore_axis_name="sub"),
        out_type=[
            jax.ShapeDtypeStruct((_B, _IMG), jnp.float32),
            jax.ShapeDtypeStruct((_B, _IMG), jnp.float32),
            jax.ShapeDtypeStruct((_B, _D), jnp.float32),
            jax.ShapeDtypeStruct((_B, _D), jnp.float32),
            jax.ShapeDtypeStruct((_B, _D), jnp.float32),
        ],
        scratch_types=[pltpu.VMEM((_BPW,), jnp.int32)] * 3 + set_types * _NBUF,
    )
    return fn(*args)


def _tc_body(gs, go, ps, po, b3, w, bl, gm, bt, out, tmp_s, tmp_o, acc):
    p = pl.program_id(0)
    t = pl.program_id(1)

    @pl.when(p == 0)
    def _pass1():
        wb = w[...]
        ts = lax.dot_general(gs[...].astype(jnp.bfloat16), wb,
                             (((1,), (1,)), ((), ())),
                             preferred_element_type=jnp.float32) + bl[...]
        to = lax.dot_general(go[...].astype(jnp.bfloat16), wb,
                             (((1,), (1,)), ((), ())),
                             preferred_element_type=jnp.float32) + bl[...]
        tmp_s[pl.ds(t * _BT, _BT), :] = ts
        tmp_o[pl.ds(t * _BT, _BT), :] = to

        @pl.when(t == 0)
        def _init():
            acc[...] = jnp.zeros_like(acc)

        delta = jnp.concatenate([
            jnp.sum(ts, axis=0, keepdims=True),
            jnp.sum(ts * ts, axis=0, keepdims=True),
            jnp.sum(to, axis=0, keepdims=True),
            jnp.sum(to * to, axis=0, keepdims=True),
        ], axis=0)
        acc[...] = acc[...] + delta

    @pl.when(p == 1)
    def _pass2():
        a = acc[...]
        inv_b = 1.0 / _B
        mean_s = a[0:1] * inv_b
        var_s = a[1:2] * inv_b - mean_s * mean_s
        mean_o = a[2:3] * inv_b
        var_o = a[3:4] * inv_b - mean_o * mean_o
        scale_s = gm[...] * lax.rsqrt(var_s + _EPS)
        shift_s = bt[...] - mean_s * scale_s
        scale_o = gm[...] * lax.rsqrt(var_o + _EPS)
        shift_o = bt[...] - mean_o * scale_o

        ts = tmp_s[pl.ds(t * _BT, _BT), :] * scale_s + shift_s
        to = tmp_o[pl.ds(t * _BT, _BT), :] * scale_o + shift_o
        psv = ps[...]
        pov = po[...]
        prod = ts * psv + to * pov + ts * to
        ones = jnp.ones((_D, 1), jnp.float32)
        rsum = lambda x: lax.dot_general(
            x, ones, (((1,), (0,)), ((), ())),
            preferred_element_type=jnp.float32)
        base = rsum(b3[...])
        ht = rsum(psv)
        tt = rsum(pov)
        isum = rsum(prod)
        sig = lambda x: jax.nn.sigmoid(_PSI * x)
        out[...] = _MULT * (sig(base) * sig(ht) * sig(tt) + 0.005 * isum)


def _tc_finish(gs, go, ps, po, b3, w, bl, gm, bt):
    return pl.pallas_call(
        _tc_body,
        grid=(2, _NT),
        in_specs=[
            pl.BlockSpec((_BT, _IMG), lambda p, t: ((1 - p) * t, 0)),
            pl.BlockSpec((_BT, _IMG), lambda p, t: ((1 - p) * t, 0)),
            pl.BlockSpec((_BT, _D), lambda p, t: (p * t, 0)),
            pl.BlockSpec((_BT, _D), lambda p, t: (p * t, 0)),
            pl.BlockSpec((_BT, _D), lambda p, t: (p * t, 0)),
            pl.BlockSpec((_D, _IMG), lambda p, t: (0, 0)),
            pl.BlockSpec((1, _D), lambda p, t: (0, 0)),
            pl.BlockSpec((1, _D), lambda p, t: (0, 0)),
            pl.BlockSpec((1, _D), lambda p, t: (0, 0)),
        ],
        out_specs=pl.BlockSpec((_BT, 1), lambda p, t: (p * t, 0)),
        out_shape=jax.ShapeDtypeStruct((_B, 1), jnp.float32),
        scratch_shapes=[
            pltpu.VMEM((_B, _D), jnp.float32),
            pltpu.VMEM((_B, _D), jnp.float32),
            pltpu.VMEM((4, _D), jnp.float32),
        ],
        compiler_params=pltpu.CompilerParams(
            dimension_semantics=("arbitrary", "arbitrary")),
    )(gs, go, ps, po, b3, w, bl, gm, bt)


def kernel(s, r, o, E_t, R_ht, R_tt, E_b, R_b, img_emb, W_lin, b_lin, gamma, beta):
    s1 = s.reshape(-1).astype(jnp.int32)
    r1 = r.reshape(-1).astype(jnp.int32)
    o1 = o.reshape(-1).astype(jnp.int32)
    r_all = jnp.concatenate([R_ht, R_tt, R_b], axis=1)
    gs, go, ps, po, b3 = _sc_gather(s1, r1, o1, E_t, r_all, E_b, img_emb)
    wb = W_lin.astype(jnp.bfloat16)
    return _tc_finish(gs, go, ps, po, b3, wb,
                      b_lin.reshape(1, _D), gamma.reshape(1, _D),
                      beta.reshape(1, _D))


# speedup vs baseline: 1.2811x; 1.2811x over previous
import jax
import jax.numpy as jnp
import numpy as np
from jax import lax
from jax.experimental import pallas as pl
from jax.experimental.pallas import tpu as pltpu
from jax.experimental.pallas import tpu_sc as plsc

_PSI = 1.0
_MULT = 20.0
_EPS = 1e-5

_B = 16384
_D = 128
_IMG = 512
_IMGP = _IMG // 2
_NW = 32
_BPW = _B // _NW
_C = 32
_NCH = _BPW // _C
_NBUF = 2

_BT = 1024
_NT = _B // _BT

_SET_KEYS = ("ets", "eto", "ebs", "ebo", "rht", "rtt", "rb", "imgfs",
             "imgfo", "si", "so")

def _sc_body(s_h, r_h, o_h, et_h, rht_h, rtt_h, rb_h, eb_h, img_h,
             gs_h, go_h, ps_h, po_h, b3_h,
             sidx, ridx, oidx, *bufs):
    wid = lax.axis_index("sub") * 2 + lax.axis_index("core")
    base = wid * _BPW
    pltpu.sync_copy(s_h.at[pl.ds(base, _BPW)], sidx)
    pltpu.sync_copy(r_h.at[pl.ds(base, _BPW)], ridx)
    pltpu.sync_copy(o_h.at[pl.ds(base, _BPW)], oidx)

    nk = len(_SET_KEYS)
    sets = [dict(zip(_SET_KEYS, bufs[i * nk:(i + 1) * nk]))
            for i in range(_NBUF)]

    def fire_in(c, st):
        sl = pl.ds(c * _C, _C)
        cs, cr, co = sidx.at[sl], ridx.at[sl], oidx.at[sl]
        return [
            pltpu.async_copy(et_h.at[cs], st["ets"], st["si"]),
            pltpu.async_copy(et_h.at[co], st["eto"], st["si"]),
            pltpu.async_copy(eb_h.at[cs], st["ebs"], st["si"]),
            pltpu.async_copy(eb_h.at[co], st["ebo"], st["si"]),
            pltpu.async_copy(rht_h.at[cr], st["rht"], st["si"]),
            pltpu.async_copy(rtt_h.at[cr], st["rtt"], st["si"]),
            pltpu.async_copy(rb_h.at[cr], st["rb"], st["si"]),
            pltpu.async_copy(img_h.at[cs], st["imgfs"], st["si"]),
            pltpu.async_copy(img_h.at[co], st["imgfo"], st["si"]),
        ]

    def compute(st):
        ets, eto, ebs, ebo = st["ets"], st["eto"], st["ebs"], st["ebo"]
        rht, rtt, rb = st["rht"], st["rtt"], st["rb"]

        def prow(i, cc):
            for j in range(_D // 16):
                ix = (i, pl.ds(j * 16, 16))
                ets[ix] = ets[ix] * rht[ix]
                eto[ix] = eto[ix] * rtt[ix]
                ebs[ix] = ebs[ix] * rb[ix] * ebo[ix]
            return cc

        lax.fori_loop(0, _C, prow, 0)

    def fire_out(c, st):
        sl = pl.ds(base + c * _C, _C)
        return [
            pltpu.async_copy(st["ets"], ps_h.at[sl], st["so"]),
            pltpu.async_copy(st["eto"], po_h.at[sl], st["so"]),
            pltpu.async_copy(st["ebs"], b3_h.at[sl], st["so"]),
            pltpu.async_copy(st["imgfs"], gs_h.at[sl], st["so"]),
            pltpu.async_copy(st["imgfo"], go_h.at[sl], st["so"]),
        ]

    in_fl = {0: fire_in(0, sets[0])}
    out_fl = [[], []]
    for c in range(_NCH):
        b = c % _NBUF
        if c + 1 < _NCH:
            nb = (c + 1) % _NBUF
            for cp in out_fl[nb]:
                cp.wait()
            out_fl[nb] = []
            in_fl[c + 1] = fire_in(c + 1, sets[nb])
        for cp in in_fl.pop(c):
            cp.wait()
        compute(sets[b])
        out_fl[b] = fire_out(c, sets[b])
    for lst in out_fl:
        for cp in lst:
            cp.wait()


def _sc_gather(*args):
    set_types = [
        pltpu.VMEM((_C, _D), jnp.float32),
        pltpu.VMEM((_C, _D), jnp.float32),
        pltpu.VMEM((_C, _D), jnp.float32),
        pltpu.VMEM((_C, _D), jnp.float32),
        pltpu.VMEM((_C, _D), jnp.float32),
        pltpu.VMEM((_C, _D), jnp.float32),
        pltpu.VMEM((_C, _D), jnp.float32),
        pltpu.VMEM((_C, _IMG), jnp.float32),
        pltpu.VMEM((_C, _IMG), jnp.float32),
        pltpu.SemaphoreType.DMA,
        pltpu.SemaphoreType.DMA,
    ]
    fn = pl.kernel(
        _sc_body,
        mesh=plsc.VectorSubcoreMesh(core_axis_name="core", subcore_axis_name="sub"),
        out_type=[
            jax.ShapeDtypeStruct((_B, _IMG), jnp.float32),
            jax.ShapeDtypeStruct((_B, _IMG), jnp.float32),
            jax.ShapeDtypeStruct((_B, _D), jnp.float32),
            jax.ShapeDtypeStruct((_B, _D), jnp.float32),
            jax.ShapeDtypeStruct((_B, _D), jnp.float32),
        ],
        scratch_types=[pltpu.VMEM((_BPW,), jnp.int32)] * 3 + set_types * _NBUF,
    )
    return fn(*args)


def _tc_body(gs, go, ps, po, b3, w, bl, gm, bt, out, tmp_s, tmp_o, acc):
    p = pl.program_id(0)
    t = pl.program_id(1)

    @pl.when(p == 0)
    def _pass1():
        wb = w[...]
        ts = lax.dot_general(gs[...].astype(jnp.bfloat16), wb,
                             (((1,), (1,)), ((), ())),
                             preferred_element_type=jnp.float32) + bl[...]
        to = lax.dot_general(go[...].astype(jnp.bfloat16), wb,
                             (((1,), (1,)), ((), ())),
                             preferred_element_type=jnp.float32) + bl[...]
        tmp_s[pl.ds(t * _BT, _BT), :] = ts
        tmp_o[pl.ds(t * _BT, _BT), :] = to

        @pl.when(t == 0)
        def _init():
            acc[...] = jnp.zeros_like(acc)

        delta = jnp.concatenate([
            jnp.sum(ts, axis=0, keepdims=True),
            jnp.sum(ts * ts, axis=0, keepdims=True),
            jnp.sum(to, axis=0, keepdims=True),
            jnp.sum(to * to, axis=0, keepdims=True),
        ], axis=0)
        acc[...] = acc[...] + delta

    @pl.when(p == 1)
    def _pass2():
        a = acc[...]
        inv_b = 1.0 / _B
        mean_s = a[0:1] * inv_b
        var_s = a[1:2] * inv_b - mean_s * mean_s
        mean_o = a[2:3] * inv_b
        var_o = a[3:4] * inv_b - mean_o * mean_o
        scale_s = gm[...] * lax.rsqrt(var_s + _EPS)
        shift_s = bt[...] - mean_s * scale_s
        scale_o = gm[...] * lax.rsqrt(var_o + _EPS)
        shift_o = bt[...] - mean_o * scale_o

        ts = tmp_s[pl.ds(t * _BT, _BT), :] * scale_s + shift_s
        to = tmp_o[pl.ds(t * _BT, _BT), :] * scale_o + shift_o
        psv = ps[...]
        pov = po[...]
        prod = ts * psv + to * pov + ts * to
        ones = jnp.ones((_D, 1), jnp.float32)
        rsum = lambda x: lax.dot_general(
            x, ones, (((1,), (0,)), ((), ())),
            preferred_element_type=jnp.float32)
        base = rsum(b3[...])
        ht = rsum(psv)
        tt = rsum(pov)
        isum = rsum(prod)
        sig = lambda x: jax.nn.sigmoid(_PSI * x)
        out[...] = _MULT * (sig(base) * sig(ht) * sig(tt) + 0.005 * isum)


def _tc_finish(gs, go, ps, po, b3, w, bl, gm, bt):
    return pl.pallas_call(
        _tc_body,
        grid=(2, _NT),
        in_specs=[
            pl.BlockSpec((_BT, _IMG), lambda p, t: ((1 - p) * t, 0)),
            pl.BlockSpec((_BT, _IMG), lambda p, t: ((1 - p) * t, 0)),
            pl.BlockSpec((_BT, _D), lambda p, t: (p * t, 0)),
            pl.BlockSpec((_BT, _D), lambda p, t: (p * t, 0)),
            pl.BlockSpec((_BT, _D), lambda p, t: (p * t, 0)),
            pl.BlockSpec((_D, _IMG), lambda p, t: (0, 0)),
            pl.BlockSpec((1, _D), lambda p, t: (0, 0)),
            pl.BlockSpec((1, _D), lambda p, t: (0, 0)),
            pl.BlockSpec((1, _D), lambda p, t: (0, 0)),
        ],
        out_specs=pl.BlockSpec((_BT, 1), lambda p, t: (p * t, 0)),
        out_shape=jax.ShapeDtypeStruct((_B, 1), jnp.float32),
        scratch_shapes=[
            pltpu.VMEM((_B, _D), jnp.float32),
            pltpu.VMEM((_B, _D), jnp.float32),
            pltpu.VMEM((4, _D), jnp.float32),
        ],
        compiler_params=pltpu.CompilerParams(
            dimension_semantics=("arbitrary", "arbitrary")),
    )(gs, go, ps, po, b3, w, bl, gm, bt)


def kernel(s, r, o, E_t, R_ht, R_tt, E_b, R_b, img_emb, W_lin, b_lin, gamma, beta):
    s1 = s.reshape(-1).astype(jnp.int32)
    r1 = r.reshape(-1).astype(jnp.int32)
    o1 = o.reshape(-1).astype(jnp.int32)
    gs, go, ps, po, b3 = _sc_gather(s1, r1, o1, E_t, R_ht, R_tt, R_b,
                                    E_b, img_emb)
    wb = W_lin.astype(jnp.bfloat16)
    return _tc_finish(gs, go, ps, po, b3, wb,
                      b_lin.reshape(1, _D), gamma.reshape(1, _D),
                      beta.reshape(1, _D))


# speedup vs baseline: 1.3445x; 1.0495x over previous
import jax
import jax.numpy as jnp
import numpy as np
from jax import lax
from jax.experimental import pallas as pl
from jax.experimental.pallas import tpu as pltpu
from jax.experimental.pallas import tpu_sc as plsc

_PSI = 1.0
_MULT = 20.0
_EPS = 1e-5

_B = 16384
_D = 128
_IMG = 512
_IMGP = _IMG // 2
_NW = 32
_BPW = _B // _NW
_C = 32
_NCH = _BPW // _C
_NBUF = 2

_BT = 2048
_NT = _B // _BT

_SET_KEYS = ("ets", "eto", "ebs", "ebo", "rht", "rtt", "rb", "imgfs",
             "imgfo", "si", "so")

def _sc_body(s_h, r_h, o_h, et_h, rht_h, rtt_h, rb_h, eb_h, img_h,
             gs_h, go_h, ps_h, po_h, b3_h,
             sidx, ridx, oidx, *bufs):
    wid = lax.axis_index("sub") * 2 + lax.axis_index("core")
    base = wid * _BPW
    pltpu.sync_copy(s_h.at[pl.ds(base, _BPW)], sidx)
    pltpu.sync_copy(r_h.at[pl.ds(base, _BPW)], ridx)
    pltpu.sync_copy(o_h.at[pl.ds(base, _BPW)], oidx)

    nk = len(_SET_KEYS)
    sets = [dict(zip(_SET_KEYS, bufs[i * nk:(i + 1) * nk]))
            for i in range(_NBUF)]

    def fire_in(c, st):
        sl = pl.ds(c * _C, _C)
        cs, cr, co = sidx.at[sl], ridx.at[sl], oidx.at[sl]
        return [
            pltpu.async_copy(et_h.at[cs], st["ets"], st["si"]),
            pltpu.async_copy(et_h.at[co], st["eto"], st["si"]),
            pltpu.async_copy(eb_h.at[cs], st["ebs"], st["si"]),
            pltpu.async_copy(eb_h.at[co], st["ebo"], st["si"]),
            pltpu.async_copy(rht_h.at[cr], st["rht"], st["si"]),
            pltpu.async_copy(rtt_h.at[cr], st["rtt"], st["si"]),
            pltpu.async_copy(rb_h.at[cr], st["rb"], st["si"]),
            pltpu.async_copy(img_h.at[cs], st["imgfs"], st["si"]),
            pltpu.async_copy(img_h.at[co], st["imgfo"], st["si"]),
        ]

    def compute(st):
        ets, eto, ebs, ebo = st["ets"], st["eto"], st["ebs"], st["ebo"]
        rht, rtt, rb = st["rht"], st["rtt"], st["rb"]

        def prow(i, cc):
            for j in range(_D // 16):
                ix = (i, pl.ds(j * 16, 16))
                ets[ix] = ets[ix] * rht[ix]
                eto[ix] = eto[ix] * rtt[ix]
                ebs[ix] = ebs[ix] * rb[ix] * ebo[ix]
            return cc

        lax.fori_loop(0, _C, prow, 0)

    def fire_out(c, st):
        sl = pl.ds(base + c * _C, _C)
        return [
            pltpu.async_copy(st["ets"], ps_h.at[sl], st["so"]),
            pltpu.async_copy(st["eto"], po_h.at[sl], st["so"]),
            pltpu.async_copy(st["ebs"], b3_h.at[sl], st["so"]),
            pltpu.async_copy(st["imgfs"], gs_h.at[sl], st["so"]),
            pltpu.async_copy(st["imgfo"], go_h.at[sl], st["so"]),
        ]

    in_fl = {0: fire_in(0, sets[0])}
    out_fl = [[], []]
    for c in range(_NCH):
        b = c % _NBUF
        if c + 1 < _NCH:
            nb = (c + 1) % _NBUF
            for cp in out_fl[nb]:
                cp.wait()
            out_fl[nb] = []
            in_fl[c + 1] = fire_in(c + 1, sets[nb])
        for cp in in_fl.pop(c):
            cp.wait()
        compute(sets[b])
        out_fl[b] = fire_out(c, sets[b])
    for lst in out_fl:
        for cp in lst:
            cp.wait()


def _sc_gather(*args):
    set_types = [
        pltpu.VMEM((_C, _D), jnp.float32),
        pltpu.VMEM((_C, _D), jnp.float32),
        pltpu.VMEM((_C, _D), jnp.float32),
        pltpu.VMEM((_C, _D), jnp.float32),
        pltpu.VMEM((_C, _D), jnp.float32),
        pltpu.VMEM((_C, _D), jnp.float32),
        pltpu.VMEM((_C, _D), jnp.float32),
        pltpu.VMEM((_C, _IMG), jnp.float32),
        pltpu.VMEM((_C, _IMG), jnp.float32),
        pltpu.SemaphoreType.DMA,
        pltpu.SemaphoreType.DMA,
    ]
    fn = pl.kernel(
        _sc_body,
        mesh=plsc.VectorSubcoreMesh(core_axis_name="core", subcore_axis_name="sub"),
        out_type=[
            jax.ShapeDtypeStruct((_B, _IMG), jnp.float32),
            jax.ShapeDtypeStruct((_B, _IMG), jnp.float32),
            jax.ShapeDtypeStruct((_B, _D), jnp.float32),
            jax.ShapeDtypeStruct((_B, _D), jnp.float32),
            jax.ShapeDtypeStruct((_B, _D), jnp.float32),
        ],
        scratch_types=[pltpu.VMEM((_BPW,), jnp.int32)] * 3 + set_types * _NBUF,
    )
    return fn(*args)


def _tc_body(gs, go, ps, po, b3, w, bl, gm, bt, out, tmp_s, tmp_o, acc):
    p = pl.program_id(0)
    t = pl.program_id(1)

    @pl.when(p == 0)
    def _pass1():
        wb = w[...]
        ts = lax.dot_general(gs[...].astype(jnp.bfloat16), wb,
                             (((1,), (1,)), ((), ())),
                             preferred_element_type=jnp.float32) + bl[...]
        to = lax.dot_general(go[...].astype(jnp.bfloat16), wb,
                             (((1,), (1,)), ((), ())),
                             preferred_element_type=jnp.float32) + bl[...]
        tmp_s[pl.ds(t * _BT, _BT), :] = ts
        tmp_o[pl.ds(t * _BT, _BT), :] = to

        @pl.when(t == 0)
        def _init():
            acc[...] = jnp.zeros_like(acc)

        delta = jnp.concatenate([
            jnp.sum(ts, axis=0, keepdims=True),
            jnp.sum(ts * ts, axis=0, keepdims=True),
            jnp.sum(to, axis=0, keepdims=True),
            jnp.sum(to * to, axis=0, keepdims=True),
        ], axis=0)
        acc[...] = acc[...] + delta

    @pl.when(p == 1)
    def _pass2():
        a = acc[...]
        inv_b = 1.0 / _B
        mean_s = a[0:1] * inv_b
        var_s = a[1:2] * inv_b - mean_s * mean_s
        mean_o = a[2:3] * inv_b
        var_o = a[3:4] * inv_b - mean_o * mean_o
        scale_s = gm[...] * lax.rsqrt(var_s + _EPS)
        shift_s = bt[...] - mean_s * scale_s
        scale_o = gm[...] * lax.rsqrt(var_o + _EPS)
        shift_o = bt[...] - mean_o * scale_o

        ts = tmp_s[pl.ds(t * _BT, _BT), :] * scale_s + shift_s
        to = tmp_o[pl.ds(t * _BT, _BT), :] * scale_o + shift_o
        psv = ps[...]
        pov = po[...]
        prod = ts * psv + to * pov + ts * to
        ones = jnp.ones((_D, 1), jnp.float32)
        rsum = lambda x: lax.dot_general(
            x, ones, (((1,), (0,)), ((), ())),
            preferred_element_type=jnp.float32)
        base = rsum(b3[...])
        ht = rsum(psv)
        tt = rsum(pov)
        isum = rsum(prod)
        sig = lambda x: jax.nn.sigmoid(_PSI * x)
        out[...] = _MULT * (sig(base) * sig(ht) * sig(tt) + 0.005 * isum)


def _tc_finish(gs, go, ps, po, b3, w, bl, gm, bt):
    return pl.pallas_call(
        _tc_body,
        grid=(2, _NT),
        in_specs=[
            pl.BlockSpec((_BT, _IMG), lambda p, t: ((1 - p) * t, 0)),
            pl.BlockSpec((_BT, _IMG), lambda p, t: ((1 - p) * t, 0)),
            pl.BlockSpec((_BT, _D), lambda p, t: (p * t, 0)),
            pl.BlockSpec((_BT, _D), lambda p, t: (p * t, 0)),
            pl.BlockSpec((_BT, _D), lambda p, t: (p * t, 0)),
            pl.BlockSpec((_D, _IMG), lambda p, t: (0, 0)),
            pl.BlockSpec((1, _D), lambda p, t: (0, 0)),
            pl.BlockSpec((1, _D), lambda p, t: (0, 0)),
            pl.BlockSpec((1, _D), lambda p, t: (0, 0)),
        ],
        out_specs=pl.BlockSpec((_BT, 1), lambda p, t: (p * t, 0)),
        out_shape=jax.ShapeDtypeStruct((_B, 1), jnp.float32),
        scratch_shapes=[
            pltpu.VMEM((_B, _D), jnp.float32),
            pltpu.VMEM((_B, _D), jnp.float32),
            pltpu.VMEM((4, _D), jnp.float32),
        ],
        compiler_params=pltpu.CompilerParams(
            dimension_semantics=("arbitrary", "arbitrary")),
    )(gs, go, ps, po, b3, w, bl, gm, bt)


def kernel(s, r, o, E_t, R_ht, R_tt, E_b, R_b, img_emb, W_lin, b_lin, gamma, beta):
    s1 = s.reshape(-1).astype(jnp.int32)
    r1 = r.reshape(-1).astype(jnp.int32)
    o1 = o.reshape(-1).astype(jnp.int32)
    gs, go, ps, po, b3 = _sc_gather(s1, r1, o1, E_t, R_ht, R_tt, R_b,
                                    E_b, img_emb)
    wb = W_lin.astype(jnp.bfloat16)
    return _tc_finish(gs, go, ps, po, b3, wb,
                      b_lin.reshape(1, _D), gamma.reshape(1, _D),
                      beta.reshape(1, _D))
